# baseline (speedup 1.0000x reference)
import functools

import jax
import jax.numpy as jnp
from jax import lax
from jax.experimental import pallas as pl
from jax.experimental.pallas import tpu as pltpu
from jax.experimental.pallas import tpu_sc as plsc

HID = 128
HEADS = 8
OUTC = HID // HEADS
NSUB = 16
C = 128
BN = 400

_f32 = jnp.float32


def _ln(t, g, b):
    mu = jnp.mean(t, axis=-1, keepdims=True)
    var = jnp.mean((t - mu) * (t - mu), axis=-1, keepdims=True)
    return (t - mu) * lax.rsqrt(var + 1e-5) * g + b


def _layer_tail(h, Wl, Wr, attf, G):
    xl = jnp.dot(h, Wl, preferred_element_type=_f32)
    xr = jnp.dot(h, Wr, preferred_element_type=_f32)
    s = xl + xr
    e = jnp.maximum(s, 0.0) + 0.2 * jnp.minimum(s, 0.0)
    logit8 = jnp.dot(e * attf, G, preferred_element_type=_f32)
    a = jnp.exp(logit8)
    ab = jnp.dot(a, G.T, preferred_element_type=_f32)
    return xl, xr, ab * xl, a


def _k1_body(x_ref, pW_ref, vec_ref, Wl_ref, Wr_ref, G_ref,
             h_ref, xl_ref, xr_ref, snum_ref, sden_ref):
    x = x_ref[...]
    t = jnp.dot(x, pW_ref[...], preferred_element_type=_f32) + vec_ref[0]
    h = jnp.maximum(_ln(t, vec_ref[1], vec_ref[2]), 0.0)
    xl, xr, snum, sden = _layer_tail(h, Wl_ref[...], Wr_ref[...], vec_ref[3],
                                     G_ref[...])
    h_ref[...] = h
    xl_ref[...] = xl
    xr_ref[...] = xr
    snum_ref[...] = snum
    sden_ref[...] = sden


def _merge(hprev, snum, sden, enum3, eden, vec, G):
    num = snum + jnp.concatenate([enum3[h] for h in range(HEADS)], axis=-1)
    den = jnp.dot(sden + eden, G.T, preferred_element_type=_f32)
    agg = num / (den + 1e-16) + vec[0]
    el = jnp.where(agg > 0, agg, jnp.exp(jnp.minimum(agg, 0.0)) - 1.0)
    return _ln(el + hprev, vec[1], vec[2])


def _k2_body(h_ref, snum_ref, sden_ref, enum_ref, eden_ref, vec_ref,
             Wl_ref, Wr_ref, G_ref,
             h1_ref, xl_ref, xr_ref, snum2_ref, sden2_ref):
    G = G_ref[...]
    h1 = _merge(h_ref[...], snum_ref[...], sden_ref[...], enum_ref[...],
                eden_ref[...], vec_ref[...], G)
    xl, xr, snum, sden = _layer_tail(h1, Wl_ref[...], Wr_ref[...], vec_ref[3], G)
    h1_ref[...] = h1
    xl_ref[...] = xl
    xr_ref[...] = xr
    snum2_ref[...] = snum
    sden2_ref[...] = sden


def _k3_body(h_ref, snum_ref, sden_ref, enum_ref, eden_ref, vec_ref, G_ref,
             z_ref, *, inv_n):
    h2 = _merge(h_ref[...], snum_ref[...], sden_ref[...], enum_ref[...],
                eden_ref[...], vec_ref[...], G_ref[...])

    @pl.when(pl.program_id(0) == 0)
    def _():
        z_ref[...] = jnp.zeros_like(z_ref)

    z_ref[...] += jnp.sum(h2, axis=0, keepdims=True) * inv_n


def _row_spec(bn, cols):
    return pl.BlockSpec((bn, cols), lambda i: (i, 0))


def _full_spec(shape):
    nd = len(shape)
    return pl.BlockSpec(shape, lambda i: (0,) * nd)


@functools.partial(jax.jit, static_argnames=())
def _noop():
    pass


def _k1_call(xt, pW, vec, Wl, Wr, G):
    N = xt.shape[0]
    grid = (N // BN,)
    out = (
        jax.ShapeDtypeStruct((N, HID), _f32),
        jax.ShapeDtypeStruct((N, HID), _f32),
        jax.ShapeDtypeStruct((N, HID), _f32),
        jax.ShapeDtypeStruct((N, HID), _f32),
        jax.ShapeDtypeStruct((N, HEADS), _f32),
    )
    return pl.pallas_call(
        _k1_body,
        grid=grid,
        in_specs=[_row_spec(BN, HID), _full_spec((HID, HID)),
                  _full_spec((4, HID)), _full_spec((HID, HID)),
                  _full_spec((HID, HID)), _full_spec((HID, HEADS))],
        out_specs=(_row_spec(BN, HID), _row_spec(BN, HID), _row_spec(BN, HID),
                   _row_spec(BN, HID), _row_spec(BN, HEADS)),
        out_shape=out,
    )(xt, pW, vec, Wl, Wr, G)


def _enum_spec(off_blk):
    return pl.BlockSpec((HEADS, BN, OUTC), lambda i: (0, off_blk + i, 0))


def _eden_spec(off_blk):
    return pl.BlockSpec((BN, HEADS), lambda i: (off_blk + i, 0))


def _k2_call(h, snum, sden, enum, eden, off, vec, Wl, Wr, G):
    N = h.shape[0]
    grid = (N // BN,)
    off_blk = off // BN
    out = (
        jax.ShapeDtypeStruct((N, HID), _f32),
        jax.ShapeDtypeStruct((N, HID), _f32),
        jax.ShapeDtypeStruct((N, HID), _f32),
        jax.ShapeDtypeStruct((N, HID), _f32),
        jax.ShapeDtypeStruct((N, HEADS), _f32),
    )
    return pl.pallas_call(
        _k2_body,
        grid=grid,
        in_specs=[_row_spec(BN, HID), _row_spec(BN, HID), _row_spec(BN, HEADS),
                  _enum_spec(off_blk), _eden_spec(off_blk),
                  _full_spec((4, HID)), _full_spec((HID, HID)),
                  _full_spec((HID, HID)), _full_spec((HID, HEADS))],
        out_specs=(_row_spec(BN, HID), _row_spec(BN, HID), _row_spec(BN, HID),
                   _row_spec(BN, HID), _row_spec(BN, HEADS)),
        out_shape=out,
    )(h, snum, sden, enum, eden, vec, Wl, Wr, G)


def _k3_call(h, snum, sden, enum, eden, off, vec, G):
    N = h.shape[0]
    grid = (N // BN,)
    off_blk = off // BN
    return pl.pallas_call(
        functools.partial(_k3_body, inv_n=1.0 / N),
        grid=grid,
        in_specs=[_row_spec(BN, HID), _row_spec(BN, HID), _row_spec(BN, HEADS),
                  _enum_spec(off_blk), _eden_spec(off_blk),
                  _full_spec((4, HID)), _full_spec((HID, HEADS))],
        out_specs=pl.BlockSpec((1, HID), lambda i: (0, 0)),
        out_shape=jax.ShapeDtypeStruct((1, HID), _f32),
        compiler_params=pltpu.CompilerParams(
            dimension_semantics=("arbitrary",)),
    )(h, snum, sden, enum, eden, vec, G)


NBUF = 6
NIDX = 3 * NBUF


@functools.lru_cache(maxsize=None)
def _make_sc_edge(N_pad, ntc, ns):
    ngraph = len(ns)
    offs = [sum(ns[:i]) for i in range(ngraph)]
    TROWS = N_pad // NSUB
    HPC = HEADS // 2
    nsteps = ntc // NBUF
    mesh = plsc.VectorSubcoreMesh(core_axis_name="c", subcore_axis_name="s")

    @functools.partial(
        pl.kernel,
        out_type=(jax.ShapeDtypeStruct((HEADS, N_pad, OUTC), _f32),
                  jax.ShapeDtypeStruct((HEADS * N_pad,), _f32)),
        mesh=mesh,
        compiler_params=pltpu.CompilerParams(use_tc_tiling_on_sc=False,
                                             needs_layout_passes=False),
        scratch_types=[
            pltpu.VMEM((ntc + OUTC,), jnp.int32),
            pltpu.VMEM((NIDX, C), jnp.int32),
            pltpu.VMEM((NIDX, C), jnp.int32),
            pltpu.VMEM((ngraph * HEADS, OUTC), _f32),
            pltpu.VMEM((NBUF, C), jnp.int32),
            pltpu.VMEM((NBUF, C), jnp.int32),
            pltpu.VMEM((NBUF * C, OUTC), _f32),
            pltpu.VMEM((NBUF * C, OUTC), _f32),
            pltpu.VMEM((NBUF, C, OUTC), _f32),
            pltpu.VMEM((NBUF, C), _f32),
            pltpu.VMEM((64, OUTC), _f32),
            pltpu.VMEM((512,), _f32),
            pltpu.VMEM_SHARED((N_pad + 8, OUTC), _f32),
            pltpu.VMEM_SHARED((N_pad + 8,), _f32),
            pltpu.SemaphoreType.DMA((NBUF,)),
            pltpu.SemaphoreType.DMA((NBUF,)),
            pltpu.SemaphoreType.DMA((NIDX,)),
        ],
    )
    def sc_edge(xl2a, xl2b, xl2c, xr2a, xr2b, xr2c, src3, dst3, gid3, att,
                enum_out, eden_out,
                gidv, scb, dcb, attv, idxl, idxr, xlb, xrb, wb, ab, zb, zb1,
                num_s, den_s, semg, sems, semi):
        xls = [xl2a, xl2b, xl2c]
        xrs = [xr2a, xr2b, xr2c]
        cid = lax.axis_index("c")
        sid = lax.axis_index("s")

        pltpu.sync_copy(gid3.at[sid], gidv)
        pltpu.sync_copy(att, attv)

        zero16 = jnp.zeros((OUTC,), _f32)
        iot = lax.iota(jnp.int32, OUTC)
        perms = [jnp.bitwise_xor(iot, 1 << k) for k in range(4)]

        def _zb_zero(i, _):
            zb[i, :] = zero16
            return 0
        lax.fori_loop(0, 64, _zb_zero, 0)

        def _zb1_zero(i, _):
            zb1[pl.ds(i * OUTC, OUTC)] = zero16
            return 0
        lax.fori_loop(0, 512 // OUTC, _zb1_zero, 0)

        my_base = sid * TROWS

        def _fire_idx(slot, ci):
            pltpu.async_copy(src3.at[sid, ci], scb.at[slot], semi.at[slot])
            pltpu.async_copy(dst3.at[sid, ci], dcb.at[slot], semi.at[slot])

        def _wait_idx(slot):
            pltpu.make_async_copy(src3.at[sid, 0], scb.at[slot],
                                  semi.at[slot]).wait()
            pltpu.make_async_copy(dst3.at[sid, 0], dcb.at[slot],
                                  semi.at[slot]).wait()

        def _head(k, _):
            hh = cid * HPC + k

            nfull, tail = divmod(TROWS, 64)
            for j in range(nfull):
                pltpu.sync_copy(zb, num_s.at[pl.ds(my_base + j * 64, 64)])
            if tail:
                pltpu.sync_copy(zb.at[pl.ds(0, tail)],
                                num_s.at[pl.ds(my_base + nfull * 64, tail)])
            nfull1, tail1 = divmod(TROWS, 512)
            for j in range(nfull1):
                pltpu.sync_copy(zb1, den_s.at[pl.ds(my_base + j * 512, 512)])
            if tail1:
                pltpu.sync_copy(zb1.at[pl.ds(0, tail1)],
                                den_s.at[pl.ds(my_base + nfull1 * 512, tail1)])

            plsc.subcore_barrier()

            def _fill_fire(b, slot, ci):
                gv = gidv[pl.ds(ci, OUTC)][0]
                roff = hh
                lim = ns[0] * HEADS - 1
                for g in range(1, ngraph):
                    roff = roff - jnp.where(gv == g, offs[g] * HEADS, 0)
                    lim = jnp.where(gv == g, ns[g] * HEADS - 1, lim)
                for j in range(C // OUTC):
                    v = scb[slot, pl.ds(j * OUTC, OUTC)]
                    idxl[b, pl.ds(j * OUTC, OUTC)] = v * HEADS + hh
                    w = dcb[slot, pl.ds(j * OUTC, OUTC)]
                    idxr[b, pl.ds(j * OUTC, OUTC)] = jnp.minimum(
                        w * HEADS + roff, lim)
                for g in range(ngraph):
                    @pl.when(gv == g)
                    def _():
                        pltpu.async_copy(xls[g].at[idxl.at[b]],
                                         xlb.at[pl.ds(b * C, C)], semg.at[b])
                        pltpu.async_copy(xrs[g].at[idxr.at[b]],
                                         xrb.at[pl.ds(b * C, C)], semg.at[b])

            def _wait_gather(b):
                pltpu.make_async_copy(xls[0].at[idxl.at[b]],
                                      xlb.at[pl.ds(b * C, C)],
                                      semg.at[b]).wait()
                pltpu.make_async_copy(xrs[0].at[idxr.at[b]],
                                      xrb.at[pl.ds(b * C, C)],
                                      semg.at[b]).wait()

            def _wait_scatter(b):
                pltpu.make_async_copy(wb.at[b], num_s.at[dcb.at[0]],
                                      sems.at[b]).wait()
                pltpu.make_async_copy(ab.at[b], den_s.at[dcb.at[0]],
                                      sems.at[b]).wait()

            npre = min(2 * NBUF, ntc)
            for ci0 in range(npre):
                _fire_idx(ci0 % NIDX, ci0)
            for b in range(min(NBUF, ntc)):
                _wait_idx(b % NIDX)
                _fill_fire(b, b % NIDX, b)

            def _step(s, _):
                for b in range(NBUF):
                    ci = s * NBUF + b
                    _wait_gather(b)

                    @pl.when(s > 0)
                    def _():
                        _wait_scatter(b)

                    gval = gidv[pl.ds(ci, OUTC)][0]
                    atth = attv[gval * HEADS + hh, :]

                    def _group(g, _):
                        acc = jnp.zeros((OUTC,), _f32)
                        for j in range(OUTC):
                            i = b * C + g * OUTC + j
                            xlv = xlb[i, :]
                            xrv = xrb[i, :]
                            sv = xlv + xrv
                            e = (jnp.maximum(sv, 0.0)
                                 + 0.2 * jnp.minimum(sv, 0.0))
                            red = jnp.sum(e * atth)
                            av = jnp.exp(jnp.full((OUTC,), red, _f32))
                            wb[b, g * OUTC + j, :] = av * xlv
                            acc = jnp.where(iot == j, av, acc)
                        ab[b, pl.ds(g * OUTC, OUTC)] = acc
                        return 0
                    lax.fori_loop(0, C // OUTC, _group, 0)

                    slot = ci % NIDX
                    pltpu.async_copy(wb.at[b], num_s.at[dcb.at[slot]],
                                     sems.at[b], add=True)
                    pltpu.async_copy(ab.at[b], den_s.at[dcb.at[slot]],
                                     sems.at[b], add=True)

                    @pl.when(ci + NBUF < ntc)
                    def _():
                        _wait_idx((ci + NBUF) % NIDX)
                        _fill_fire(b, (ci + NBUF) % NIDX, ci + NBUF)

                    @pl.when(ci + 2 * NBUF < ntc)
                    def _():
                        _fire_idx((ci + 2 * NBUF) % NIDX, ci + 2 * NBUF)
                return 0
            lax.fori_loop(0, nsteps, _step, 0)

            for b in range(NBUF):
                _wait_scatter(b)

            plsc.subcore_barrier()

            pltpu.sync_copy(num_s.at[pl.ds(my_base, TROWS)],
                            enum_out.at[hh, pl.ds(my_base, TROWS)])
            pltpu.sync_copy(den_s.at[pl.ds(my_base, TROWS)],
                            eden_out.at[pl.ds(hh * N_pad + my_base, TROWS)])
            return 0

        lax.fori_loop(0, HPC, _head, 0)

    return sc_edge


def _prep_edges(Ns, edges):
    N_tot = sum(Ns)
    N_pad = -(-N_tot // (NSUB * 8)) * (NSUB * 8)
    offs = [sum(Ns[:i]) for i in range(len(Ns))]

    srcs, dsts, gids = [], [], []
    for g, e in enumerate(edges):
        E = e.shape[1]
        E_pad = -(-E // C) * C
        srcs.append(jnp.concatenate(
            [e[0], jnp.zeros((E_pad - E,), jnp.int32)]))
        dsts.append(jnp.concatenate(
            [e[1] + offs[g], jnp.full((E_pad - E,), N_pad, jnp.int32)]))
        gids.append(jnp.full((E_pad // C,), g, jnp.int32))
    src = jnp.concatenate(srcs)
    dst = jnp.concatenate(dsts)
    gid = jnp.concatenate(gids)

    nchunks = src.shape[0] // C
    ntc = -(-nchunks // NSUB)
    ntc = -(-ntc // NBUF) * NBUF
    tot = NSUB * ntc
    src = jnp.concatenate(
        [src, jnp.zeros(((tot - nchunks) * C,), jnp.int32)])
    dst = jnp.concatenate(
        [dst, jnp.full(((tot - nchunks) * C,), N_pad, jnp.int32)])
    gid = jnp.concatenate([gid, jnp.zeros((tot - nchunks,), jnp.int32)])
    src3 = src.reshape(NSUB, ntc, C)
    dst3 = dst.reshape(NSUB, ntc, C)
    gid3 = jnp.pad(gid.reshape(NSUB, ntc), ((0, 0), (0, OUTC)))
    return N_pad, ntc, src3, dst3, gid3


def _sc_edge_merged(Ns, N_pad, ntc, src3, dst3, gid3, xls, xrs, atts):
    tabs_l = [x.reshape(-1, OUTC) for x in xls]
    tabs_r = [x.reshape(-1, OUTC) for x in xrs]
    att = jnp.concatenate(atts)
    enum, eden = _make_sc_edge(N_pad, ntc, tuple(Ns))(
        *tabs_l, *tabs_r, src3, dst3, gid3, att)
    eden_t = jnp.transpose(eden.reshape(HEADS, N_pad), (1, 0))
    return enum, eden_t


def kernel(gene_x, meth_x, mirna_x, gene_edge, cpg_edge, mirna_edge,
           gene_params, cpg_params, mirna_params):
    B = gene_x.shape[0]
    G = (jnp.arange(HID, dtype=jnp.int32)[:, None] // OUTC ==
         jnp.arange(HEADS, dtype=jnp.int32)[None, :]).astype(_f32)

    xs = [gene_x, meth_x, mirna_x]
    edges = [gene_edge, cpg_edge, mirna_edge]
    params = [gene_params, cpg_params, mirna_params]
    Ns = [x.shape[1] for x in xs]

    vecs1, vecs2, vecs3 = [], [], []
    for p in params:
        l0, l1 = p['layers'][0], p['layers'][1]
        vecs1.append(jnp.stack([p['pb'], p['pg'], p['pB'],
                                l0['att'].reshape(HID)]))
        vecs2.append(jnp.stack([l0['bias'], l0['g'], l0['b'],
                                l1['att'].reshape(HID)]))
        vecs3.append(jnp.stack([l1['bias'], l1['g'], l1['b'],
                                jnp.zeros((HID,), _f32)]))

    N_pad, ntc, src3, dst3, gid3 = _prep_edges(Ns, edges)
    offs = [sum(Ns[:i]) for i in range(len(Ns))]

    st1 = []
    for x, p, v1 in zip(xs, params, vecs1):
        l0 = p['layers'][0]
        st1.append(_k1_call(jnp.transpose(x), p['pW'], v1,
                            l0['Wl'], l0['Wr'], G))
    en1, ed1 = _sc_edge_merged(
        Ns, N_pad, ntc, src3, dst3, gid3,
        [s[1] for s in st1], [s[2] for s in st1],
        [p['layers'][0]['att'] for p in params])
    st2 = []
    for (h0, _, _, snum, sden), off, p, v2 in zip(st1, offs, params, vecs2):
        l1 = p['layers'][1]
        st2.append(_k2_call(h0, snum, sden, en1, ed1, off, v2,
                            l1['Wl'], l1['Wr'], G))
    en2, ed2 = _sc_edge_merged(
        Ns, N_pad, ntc, src3, dst3, gid3,
        [s[1] for s in st2], [s[2] for s in st2],
        [p['layers'][1]['att'] for p in params])
    zs = []
    for (h1, _, _, snum2, sden2), off, v3 in zip(st2, offs, vecs3):
        zs.append(_k3_call(h1, snum2, sden2, en2, ed2, off, v3, G))
    return (jnp.broadcast_to(zs[0], (B, HID)),
            jnp.broadcast_to(zs[1], (B, HID)),
            jnp.broadcast_to(zs[2], (B, HID)))


# speedup vs baseline: 1.2406x; 1.2406x over previous
import functools

import jax
import jax.numpy as jnp
from jax import lax
from jax.experimental import pallas as pl
from jax.experimental.pallas import tpu as pltpu
from jax.experimental.pallas import tpu_sc as plsc

HID = 128
HEADS = 8
OUTC = HID // HEADS
NSUB = 16
C = 128
BN = 400

_f32 = jnp.float32


def _ln(t, g, b):
    mu = jnp.mean(t, axis=-1, keepdims=True)
    var = jnp.mean((t - mu) * (t - mu), axis=-1, keepdims=True)
    return (t - mu) * lax.rsqrt(var + 1e-5) * g + b


def _layer_tail(h, Wl, Wr, attf, G):
    xl = jnp.dot(h, Wl, preferred_element_type=_f32)
    xr = jnp.dot(h, Wr, preferred_element_type=_f32)
    s = xl + xr
    e = jnp.maximum(s, 0.0) + 0.2 * jnp.minimum(s, 0.0)
    logit8 = jnp.dot(e * attf, G, preferred_element_type=_f32)
    a = jnp.exp(logit8)
    ab = jnp.dot(a, G.T, preferred_element_type=_f32)
    return xl, xr, ab * xl, a


def _k1_body(x_ref, pW_ref, vec_ref, Wl_ref, Wr_ref, G_ref,
             h_ref, xl_ref, xr_ref, snum_ref, sden_ref):
    x = x_ref[...]
    t = jnp.dot(x, pW_ref[...], preferred_element_type=_f32) + vec_ref[0]
    h = jnp.maximum(_ln(t, vec_ref[1], vec_ref[2]), 0.0)
    xl, xr, snum, sden = _layer_tail(h, Wl_ref[...], Wr_ref[...], vec_ref[3],
                                     G_ref[...])
    h_ref[...] = h
    xl_ref[...] = xl
    xr_ref[...] = xr
    snum_ref[...] = snum
    sden_ref[...] = sden


def _merge(hprev, snum, sden, enum3, eden, vec, G):
    num = snum + jnp.concatenate([enum3[h] for h in range(HEADS)], axis=-1)
    den = jnp.dot(sden + eden, G.T, preferred_element_type=_f32)
    agg = num / (den + 1e-16) + vec[0]
    el = jnp.where(agg > 0, agg, jnp.exp(jnp.minimum(agg, 0.0)) - 1.0)
    return _ln(el + hprev, vec[1], vec[2])


def _k2_body(h_ref, snum_ref, sden_ref, enum_ref, eden_ref, vec_ref,
             Wl_ref, Wr_ref, G_ref,
             h1_ref, xl_ref, xr_ref, snum2_ref, sden2_ref):
    G = G_ref[...]
    h1 = _merge(h_ref[...], snum_ref[...], sden_ref[...], enum_ref[...],
                eden_ref[...], vec_ref[...], G)
    xl, xr, snum, sden = _layer_tail(h1, Wl_ref[...], Wr_ref[...], vec_ref[3], G)
    h1_ref[...] = h1
    xl_ref[...] = xl
    xr_ref[...] = xr
    snum2_ref[...] = snum
    sden2_ref[...] = sden


def _k3_body(h_ref, snum_ref, sden_ref, enum_ref, eden_ref, vec_ref, G_ref,
             z_ref, *, inv_n):
    h2 = _merge(h_ref[...], snum_ref[...], sden_ref[...], enum_ref[...],
                eden_ref[...], vec_ref[...], G_ref[...])

    @pl.when(pl.program_id(0) == 0)
    def _():
        z_ref[...] = jnp.zeros_like(z_ref)

    z_ref[...] += jnp.sum(h2, axis=0, keepdims=True) * inv_n


def _row_spec(bn, cols):
    return pl.BlockSpec((bn, cols), lambda i: (i, 0))


def _full_spec(shape):
    nd = len(shape)
    return pl.BlockSpec(shape, lambda i: (0,) * nd)


@functools.partial(jax.jit, static_argnames=())
def _noop():
    pass


def _k1_call(xt, pW, vec, Wl, Wr, G):
    N = xt.shape[0]
    grid = (N // BN,)
    out = (
        jax.ShapeDtypeStruct((N, HID), _f32),
        jax.ShapeDtypeStruct((N, HID), _f32),
        jax.ShapeDtypeStruct((N, HID), _f32),
        jax.ShapeDtypeStruct((N, HID), _f32),
        jax.ShapeDtypeStruct((N, HEADS), _f32),
    )
    return pl.pallas_call(
        _k1_body,
        grid=grid,
        in_specs=[_row_spec(BN, HID), _full_spec((HID, HID)),
                  _full_spec((4, HID)), _full_spec((HID, HID)),
                  _full_spec((HID, HID)), _full_spec((HID, HEADS))],
        out_specs=(_row_spec(BN, HID), _row_spec(BN, HID), _row_spec(BN, HID),
                   _row_spec(BN, HID), _row_spec(BN, HEADS)),
        out_shape=out,
    )(xt, pW, vec, Wl, Wr, G)


def _enum_spec(off_blk):
    return pl.BlockSpec((HEADS, BN, OUTC), lambda i: (0, off_blk + i, 0))


def _eden_spec(off_blk):
    return pl.BlockSpec((BN, HEADS), lambda i: (off_blk + i, 0))


def _k2_call(h, snum, sden, enum, eden, off, vec, Wl, Wr, G):
    N = h.shape[0]
    grid = (N // BN,)
    off_blk = off // BN
    out = (
        jax.ShapeDtypeStruct((N, HID), _f32),
        jax.ShapeDtypeStruct((N, HID), _f32),
        jax.ShapeDtypeStruct((N, HID), _f32),
        jax.ShapeDtypeStruct((N, HID), _f32),
        jax.ShapeDtypeStruct((N, HEADS), _f32),
    )
    return pl.pallas_call(
        _k2_body,
        grid=grid,
        in_specs=[_row_spec(BN, HID), _row_spec(BN, HID), _row_spec(BN, HEADS),
                  _enum_spec(off_blk), _eden_spec(off_blk),
                  _full_spec((4, HID)), _full_spec((HID, HID)),
                  _full_spec((HID, HID)), _full_spec((HID, HEADS))],
        out_specs=(_row_spec(BN, HID), _row_spec(BN, HID), _row_spec(BN, HID),
                   _row_spec(BN, HID), _row_spec(BN, HEADS)),
        out_shape=out,
    )(h, snum, sden, enum, eden, vec, Wl, Wr, G)


def _k3_call(h, snum, sden, enum, eden, off, vec, G):
    N = h.shape[0]
    grid = (N // BN,)
    off_blk = off // BN
    return pl.pallas_call(
        functools.partial(_k3_body, inv_n=1.0 / N),
        grid=grid,
        in_specs=[_row_spec(BN, HID), _row_spec(BN, HID), _row_spec(BN, HEADS),
                  _enum_spec(off_blk), _eden_spec(off_blk),
                  _full_spec((4, HID)), _full_spec((HID, HEADS))],
        out_specs=pl.BlockSpec((1, HID), lambda i: (0, 0)),
        out_shape=jax.ShapeDtypeStruct((1, HID), _f32),
        compiler_params=pltpu.CompilerParams(
            dimension_semantics=("arbitrary",)),
    )(h, snum, sden, enum, eden, vec, G)


NBUF = 4
NIDX = 3 * NBUF


@functools.lru_cache(maxsize=None)
def _make_sc_edge(N_pad, ntc, ns):
    ngraph = len(ns)
    offs = [sum(ns[:i]) for i in range(ngraph)]
    TROWS = N_pad // NSUB
    HPC = HEADS // 2
    nsteps = ntc // NBUF
    mesh = plsc.VectorSubcoreMesh(core_axis_name="c", subcore_axis_name="s")

    @functools.partial(
        pl.kernel,
        out_type=(jax.ShapeDtypeStruct((HEADS, N_pad, OUTC), _f32),
                  jax.ShapeDtypeStruct((HEADS * N_pad,), _f32)),
        mesh=mesh,
        compiler_params=pltpu.CompilerParams(use_tc_tiling_on_sc=False,
                                             needs_layout_passes=False),
        scratch_types=[
            pltpu.VMEM((ntc + OUTC,), jnp.int32),
            pltpu.VMEM((NIDX, C), jnp.int32),
            pltpu.VMEM((NIDX, C), jnp.int32),
            pltpu.VMEM((ngraph * HEADS, OUTC), _f32),
            pltpu.VMEM((NBUF, C), jnp.int32),
            pltpu.VMEM((NBUF, C), jnp.int32),
            pltpu.VMEM((NBUF * C, OUTC), _f32),
            pltpu.VMEM((NBUF * C, OUTC), _f32),
            pltpu.VMEM((NBUF, C, OUTC), _f32),
            pltpu.VMEM((NBUF, C), _f32),
            pltpu.VMEM((64, OUTC), _f32),
            pltpu.VMEM((512,), _f32),
            pltpu.VMEM_SHARED((N_pad + 8, OUTC), _f32),
            pltpu.VMEM_SHARED((N_pad + 8,), _f32),
            pltpu.SemaphoreType.DMA((NBUF,)),
            pltpu.SemaphoreType.DMA((NBUF,)),
            pltpu.SemaphoreType.DMA((NIDX,)),
        ],
    )
    def sc_edge(xl2a, xl2b, xl2c, xr2a, xr2b, xr2c, src3, dst3, gid3, att,
                enum_out, eden_out,
                gidv, scb, dcb, attv, idxl, idxr, xlb, xrb, wb, ab, zb, zb1,
                num_s, den_s, semg, sems, semi):
        xls = [xl2a, xl2b, xl2c]
        xrs = [xr2a, xr2b, xr2c]
        cid = lax.axis_index("c")
        sid = lax.axis_index("s")

        pltpu.sync_copy(gid3.at[sid], gidv)
        pltpu.sync_copy(att, attv)

        zero16 = jnp.zeros((OUTC,), _f32)
        iot = lax.iota(jnp.int32, OUTC)
        perms = [jnp.bitwise_xor(iot, 1 << k) for k in range(4)]

        def _zb_zero(i, _):
            zb[i, :] = zero16
            return 0
        lax.fori_loop(0, 64, _zb_zero, 0)

        def _zb1_zero(i, _):
            zb1[pl.ds(i * OUTC, OUTC)] = zero16
            return 0
        lax.fori_loop(0, 512 // OUTC, _zb1_zero, 0)

        my_base = sid * TROWS

        def _fire_idx(slot, ci):
            pltpu.async_copy(src3.at[sid, ci], scb.at[slot], semi.at[slot])
            pltpu.async_copy(dst3.at[sid, ci], dcb.at[slot], semi.at[slot])

        def _wait_idx(slot):
            pltpu.make_async_copy(src3.at[sid, 0], scb.at[slot],
                                  semi.at[slot]).wait()
            pltpu.make_async_copy(dst3.at[sid, 0], dcb.at[slot],
                                  semi.at[slot]).wait()

        def _head(k, _):
            hh = cid * HPC + k

            nfull, tail = divmod(TROWS, 64)
            for j in range(nfull):
                pltpu.sync_copy(zb, num_s.at[pl.ds(my_base + j * 64, 64)])
            if tail:
                pltpu.sync_copy(zb.at[pl.ds(0, tail)],
                                num_s.at[pl.ds(my_base + nfull * 64, tail)])
            nfull1, tail1 = divmod(TROWS, 512)
            for j in range(nfull1):
                pltpu.sync_copy(zb1, den_s.at[pl.ds(my_base + j * 512, 512)])
            if tail1:
                pltpu.sync_copy(zb1.at[pl.ds(0, tail1)],
                                den_s.at[pl.ds(my_base + nfull1 * 512, tail1)])

            plsc.subcore_barrier()

            def _fill_fire(b, slot, ci):
                gv = gidv[pl.ds(ci, OUTC)][0]
                roff = hh
                lim = ns[0] * HEADS - 1
                for g in range(1, ngraph):
                    roff = roff - jnp.where(gv == g, offs[g] * HEADS, 0)
                    lim = jnp.where(gv == g, ns[g] * HEADS - 1, lim)
                for j in range(C // OUTC):
                    v = scb[slot, pl.ds(j * OUTC, OUTC)]
                    idxl[b, pl.ds(j * OUTC, OUTC)] = v * HEADS + hh
                    w = dcb[slot, pl.ds(j * OUTC, OUTC)]
                    idxr[b, pl.ds(j * OUTC, OUTC)] = jnp.minimum(
                        w * HEADS + roff, lim)
                for g in range(ngraph):
                    @pl.when(gv == g)
                    def _():
                        pltpu.async_copy(xls[g].at[idxl.at[b]],
                                         xlb.at[pl.ds(b * C, C)], semg.at[b])
                        pltpu.async_copy(xrs[g].at[idxr.at[b]],
                                         xrb.at[pl.ds(b * C, C)], semg.at[b])

            def _wait_gather(b):
                pltpu.make_async_copy(xls[0].at[idxl.at[b]],
                                      xlb.at[pl.ds(b * C, C)],
                                      semg.at[b]).wait()
                pltpu.make_async_copy(xrs[0].at[idxr.at[b]],
                                      xrb.at[pl.ds(b * C, C)],
                                      semg.at[b]).wait()

            def _wait_scatter(b):
                pltpu.make_async_copy(wb.at[b], num_s.at[dcb.at[0]],
                                      sems.at[b]).wait()
                pltpu.make_async_copy(ab.at[b], den_s.at[dcb.at[0]],
                                      sems.at[b]).wait()

            npre = min(2 * NBUF, ntc)
            for ci0 in range(npre):
                _fire_idx(ci0 % NIDX, ci0)
            for b in range(min(NBUF, ntc)):
                _wait_idx(b % NIDX)
                _fill_fire(b, b % NIDX, b)

            def _step(s, _):
                for b in range(NBUF):
                    ci = s * NBUF + b
                    _wait_gather(b)

                    @pl.when(s > 0)
                    def _():
                        _wait_scatter(b)

                    gval = gidv[pl.ds(ci, OUTC)][0]
                    atth = attv[gval * HEADS + hh, :]

                    def _group(g, _):
                        acc = jnp.zeros((OUTC,), _f32)
                        for j in range(OUTC):
                            i = b * C + g * OUTC + j
                            xlv = xlb[i, :]
                            xrv = xrb[i, :]
                            sv = xlv + xrv
                            e = (jnp.maximum(sv, 0.0)
                                 + 0.2 * jnp.minimum(sv, 0.0))
                            red = jnp.sum(e * atth)
                            av = jnp.exp(jnp.full((OUTC,), red, _f32))
                            wb[b, g * OUTC + j, :] = av * xlv
                            acc = jnp.where(iot == j, av, acc)
                        ab[b, pl.ds(g * OUTC, OUTC)] = acc
                        return 0
                    lax.fori_loop(0, C // OUTC, _group, 0)

                    slot = ci % NIDX
                    pltpu.async_copy(wb.at[b], num_s.at[dcb.at[slot]],
                                     sems.at[b], add=True)
                    pltpu.async_copy(ab.at[b], den_s.at[dcb.at[slot]],
                                     sems.at[b], add=True)

                    @pl.when(ci + NBUF < ntc)
                    def _():
                        _wait_idx((ci + NBUF) % NIDX)
                        _fill_fire(b, (ci + NBUF) % NIDX, ci + NBUF)

                    @pl.when(ci + 2 * NBUF < ntc)
                    def _():
                        _fire_idx((ci + 2 * NBUF) % NIDX, ci + 2 * NBUF)
                return 0
            lax.fori_loop(0, nsteps, _step, 0)

            for b in range(NBUF):
                _wait_scatter(b)

            plsc.subcore_barrier()

            pltpu.sync_copy(num_s.at[pl.ds(my_base, TROWS)],
                            enum_out.at[hh, pl.ds(my_base, TROWS)])
            pltpu.sync_copy(den_s.at[pl.ds(my_base, TROWS)],
                            eden_out.at[pl.ds(hh * N_pad + my_base, TROWS)])
            return 0

        lax.fori_loop(0, HPC, _head, 0)

    return sc_edge


def _prep_edges(Ns, edges):
    N_tot = sum(Ns)
    N_pad = -(-N_tot // (NSUB * 8)) * (NSUB * 8)
    offs = [sum(Ns[:i]) for i in range(len(Ns))]

    srcs, dsts, gids = [], [], []
    for g, e in enumerate(edges):
        E = e.shape[1]
        E_pad = -(-E // C) * C
        srcs.append(jnp.concatenate(
            [e[0], jnp.zeros((E_pad - E,), jnp.int32)]))
        dsts.append(jnp.concatenate(
            [e[1] + offs[g], jnp.full((E_pad - E,), N_pad, jnp.int32)]))
        gids.append(jnp.full((E_pad // C,), g, jnp.int32))
    src = jnp.concatenate(srcs)
    dst = jnp.concatenate(dsts)
    gid = jnp.concatenate(gids)

    nchunks = src.shape[0] // C
    ntc = -(-nchunks // NSUB)
    ntc = -(-ntc // NBUF) * NBUF
    tot = NSUB * ntc
    src = jnp.concatenate(
        [src, jnp.zeros(((tot - nchunks) * C,), jnp.int32)])
    dst = jnp.concatenate(
        [dst, jnp.full(((tot - nchunks) * C,), N_pad, jnp.int32)])
    gid = jnp.concatenate([gid, jnp.zeros((tot - nchunks,), jnp.int32)])
    src3 = src.reshape(NSUB, ntc, C)
    dst3 = dst.reshape(NSUB, ntc, C)
    gid3 = jnp.pad(gid.reshape(NSUB, ntc), ((0, 0), (0, OUTC)))
    return N_pad, ntc, src3, dst3, gid3


def _sc_edge_merged(Ns, N_pad, ntc, src3, dst3, gid3, xls, xrs, atts):
    tabs_l = [x.reshape(-1, OUTC) for x in xls]
    tabs_r = [x.reshape(-1, OUTC) for x in xrs]
    att = jnp.concatenate(atts)
    enum, eden = _make_sc_edge(N_pad, ntc, tuple(Ns))(
        *tabs_l, *tabs_r, src3, dst3, gid3, att)
    eden_t = jnp.transpose(eden.reshape(HEADS, N_pad), (1, 0))
    return enum, eden_t


def kernel(gene_x, meth_x, mirna_x, gene_edge, cpg_edge, mirna_edge,
           gene_params, cpg_params, mirna_params):
    B = gene_x.shape[0]
    G = (jnp.arange(HID, dtype=jnp.int32)[:, None] // OUTC ==
         jnp.arange(HEADS, dtype=jnp.int32)[None, :]).astype(_f32)

    xs = [gene_x, meth_x, mirna_x]
    edges = [gene_edge, cpg_edge, mirna_edge]
    params = [gene_params, cpg_params, mirna_params]
    Ns = [x.shape[1] for x in xs]

    vecs1, vecs2, vecs3 = [], [], []
    for p in params:
        l0, l1 = p['layers'][0], p['layers'][1]
        vecs1.append(jnp.stack([p['pb'], p['pg'], p['pB'],
                                l0['att'].reshape(HID)]))
        vecs2.append(jnp.stack([l0['bias'], l0['g'], l0['b'],
                                l1['att'].reshape(HID)]))
        vecs3.append(jnp.stack([l1['bias'], l1['g'], l1['b'],
                                jnp.zeros((HID,), _f32)]))

    N_pad, ntc, src3, dst3, gid3 = _prep_edges(Ns, edges)
    offs = [sum(Ns[:i]) for i in range(len(Ns))]

    st1 = []
    for x, p, v1 in zip(xs, params, vecs1):
        l0 = p['layers'][0]
        st1.append(_k1_call(jnp.transpose(x), p['pW'], v1,
                            l0['Wl'], l0['Wr'], G))
    en1, ed1 = _sc_edge_merged(
        Ns, N_pad, ntc, src3, dst3, gid3,
        [s[1] for s in st1], [s[2] for s in st1],
        [p['layers'][0]['att'] for p in params])
    st2 = []
    for (h0, _, _, snum, sden), off, p, v2 in zip(st1, offs, params, vecs2):
        l1 = p['layers'][1]
        st2.append(_k2_call(h0, snum, sden, en1, ed1, off, v2,
                            l1['Wl'], l1['Wr'], G))
    en2, ed2 = _sc_edge_merged(
        Ns, N_pad, ntc, src3, dst3, gid3,
        [s[1] for s in st2], [s[2] for s in st2],
        [p['layers'][1]['att'] for p in params])
    zs = []
    for (h1, _, _, snum2, sden2), off, v3 in zip(st2, offs, vecs3):
        zs.append(_k3_call(h1, snum2, sden2, en2, ed2, off, v3, G))
    return (jnp.broadcast_to(zs[0], (B, HID)),
            jnp.broadcast_to(zs[1], (B, HID)),
            jnp.broadcast_to(zs[2], (B, HID)))
